# baseline (speedup 1.0000x reference)
import jax
import jax.numpy as jnp
from jax import lax
from jax.experimental import pallas as pl
from jax.experimental.pallas import tpu as pltpu
from jax.experimental.pallas import tpu_sc as plsc

INPUT_DIM = 3
INDEX_WIDTH = 100
FEATURE_SIZE = 64
BATCH = 16384
ROWS = INDEX_WIDTH ** INPUT_DIM

_INFO = plsc.get_sparse_core_info()
_NC, _NS, _L = _INFO.num_cores, _INFO.num_subcores, _INFO.num_lanes
_NW = _NC * _NS
_BPW = BATCH // _NW


def _lookup_body(x0_hbm, x1_hbm, x2_hbm, table_hbm, out_hbm,
                 c0, c1, c2, rows_v, sem):
    wid = lax.axis_index("s") * _NC + lax.axis_index("c")
    base = wid * _BPW

    pltpu.sync_copy(x0_hbm.at[pl.ds(base, _BPW)], c0)
    pltpu.sync_copy(x1_hbm.at[pl.ds(base, _BPW)], c1)
    pltpu.sync_copy(x2_hbm.at[pl.ds(base, _BPW)], c2)

    scale = jnp.float32(INDEX_WIDTH)
    iota = lax.iota(jnp.int32, _L)
    zero16 = jnp.zeros((_L,), jnp.int32)

    def fire(i, carry):
        s = pl.ds(i * _L, _L)
        lin = (c0[s] * scale).astype(jnp.int32)
        lin += (c1[s] * scale).astype(jnp.int32) * INDEX_WIDTH
        lin += (c2[s] * scale).astype(jnp.int32) * (INDEX_WIDTH * INDEX_WIDTH)
        for l in range(_L):
            r = jnp.sum(jnp.where(iota == l, lin, zero16))
            pltpu.make_async_copy(
                table_hbm.at[pl.ds(r, 1), :],
                rows_v.at[pl.ds(i * _L + l, 1), :],
                sem,
            ).start()
        return carry

    lax.fori_loop(0, _BPW // _L, fire, 0)

    pltpu.make_async_copy(
        table_hbm.at[pl.ds(0, _BPW), :], rows_v, sem
    ).wait()

    pltpu.sync_copy(rows_v, out_hbm.at[pl.ds(base, _BPW), :])


@jax.jit
def _lookup(x0, x1, x2, table):
    mesh = plsc.VectorSubcoreMesh(core_axis_name="c", subcore_axis_name="s")
    return pl.kernel(
        _lookup_body,
        out_type=jax.ShapeDtypeStruct((BATCH, FEATURE_SIZE), jnp.float32),
        mesh=mesh,
        scratch_types=[
            pltpu.VMEM((_BPW,), jnp.float32),
            pltpu.VMEM((_BPW,), jnp.float32),
            pltpu.VMEM((_BPW,), jnp.float32),
            pltpu.VMEM((_BPW, FEATURE_SIZE), jnp.float32),
            pltpu.SemaphoreType.DMA,
        ],
        compiler_params=pltpu.CompilerParams(needs_layout_passes=False),
    )(x0, x1, x2, table)


def kernel(indices, table):
    return _lookup(indices[:, 0], indices[:, 1], indices[:, 2], table)
